# baseline (speedup 1.0000x reference)
import functools

import jax
import jax.numpy as jnp
from jax import lax
from jax.experimental import pallas as pl
from jax.experimental.pallas import tpu as pltpu
from jax.experimental.pallas import tpu_sc as plsc

NUM_PROTOTYPES = 1024
DIM = 32
N_TOKENS = 16384
N_GRAPHS = 1024

TOK_BLOCK = 2048
N_BLOCKS = N_TOKENS // TOK_BLOCK


def _assign_body(e_ref, c_ref, idx_ref):
    e = e_ref[...]
    c = c_ref[...]
    a2 = jnp.sum(e * e, axis=1, keepdims=True)
    b2 = jnp.sum(c * c, axis=1)[None, :]
    prod2 = lax.dot_general(e, c * -2.0, (((1,), (1,)), ((), ())),
                            preferred_element_type=jnp.float32)
    d2 = (a2 + b2) + prod2
    e2 = jnp.maximum(d2, 1e-12)
    m2 = jnp.min(e2, axis=1, keepdims=True)
    dmin = jnp.sqrt(m2)
    mi = lax.bitcast_convert_type(m2, jnp.int32)
    B = m2
    for k in range(1, 6):
        cand = lax.bitcast_convert_type(mi + k, jnp.float32)
        B = jnp.where(jnp.sqrt(cand) == dmin, cand, B)
    ii = lax.broadcasted_iota(jnp.int32, d2.shape, 1)
    idx = jnp.min(jnp.where(e2 <= B, ii, NUM_PROTOTYPES), axis=1)
    idx_ref[0, 0, :] = idx


def _assign(le_embeddings, prototype_vectors):
    return pl.pallas_call(
        _assign_body,
        grid=(N_BLOCKS,),
        in_specs=[
            pl.BlockSpec((TOK_BLOCK, DIM), lambda i: (i, 0)),
            pl.BlockSpec((NUM_PROTOTYPES, DIM), lambda i: (0, 0)),
        ],
        out_specs=pl.BlockSpec((1, 1, TOK_BLOCK), lambda i: (i, 0, 0)),
        out_shape=jax.ShapeDtypeStruct((N_BLOCKS, 1, TOK_BLOCK), jnp.int32),
    )(le_embeddings, prototype_vectors)


L = 16
N_WORKERS = 32
ROWS_PER_TILE = N_GRAPHS // N_WORKERS
OUT_PER_TILE = ROWS_PER_TILE * NUM_PROTOTYPES
N_CHUNKS = N_TOKENS // L
COLS_CHUNKS = NUM_PROTOTYPES // L

@functools.cache
def _make_scatter():
    mesh = plsc.VectorSubcoreMesh(core_axis_name="c", subcore_axis_name="s")
    return functools.partial(
        pl.kernel,
        mesh=mesh,
        compiler_params=pltpu.CompilerParams(needs_layout_passes=False),
        out_type=jax.ShapeDtypeStruct((N_GRAPHS * NUM_PROTOTYPES,), jnp.float32),
        scratch_types=[
            pltpu.VMEM((N_TOKENS,), jnp.int32),
            pltpu.VMEM((N_TOKENS,), jnp.int32),
            pltpu.VMEM((OUT_PER_TILE,), jnp.float32),
            pltpu.VMEM((ROWS_PER_TILE,), jnp.float32),
        ],
    )(_scatter_body)


def _lower_bounds(bel_v, targets):
    def body(k, lo):
        nxt = lo + (jnp.int32(N_TOKENS) >> k)
        idx = jnp.minimum(nxt, N_TOKENS - 1)
        v = plsc.load_gather(bel_v, [idx])
        take = (nxt <= N_TOKENS - 1) & (v < targets)
        return jnp.where(take, nxt, lo)

    lo0 = jnp.full((L,), -1, jnp.int32)
    return lax.fori_loop(0, 15, body, lo0) + 1


def _scatter_body(zero_hbm, bel_hbm, idx_hbm, out_hbm, bel_v, idx_v, rows_v,
                  base_v):
    wid = lax.axis_index("s") * 2 + lax.axis_index("c")
    g0 = wid * ROWS_PER_TILE

    pltpu.sync_copy(zero_hbm, rows_v)
    pltpu.sync_copy(bel_hbm, bel_v)
    pltpu.sync_copy(idx_hbm, idx_v)

    zeros = jnp.zeros((L,), jnp.float32)
    ones = jnp.ones((L,), jnp.float32)
    neg = jnp.full((L,), -jnp.inf, jnp.float32)

    base_v[pl.ds(0, L)] = neg
    base_v[pl.ds(L, L)] = neg

    lanes = lax.iota(jnp.int32, 16)
    bounds = _lower_bounds(bel_v, g0 + lanes * ROWS_PER_TILE)
    start = jnp.max(jnp.where(lanes == 0, bounds, 0))
    end = jnp.max(jnp.where(lanes == 1, bounds, 0))

    def scan_body(c, carry):
        b = bel_v[pl.ds(c * L, L)]
        i = idx_v[pl.ds(c * L, L)]
        r = b - g0
        m = (r >= 0) & (r < ROWS_PER_TILE)
        rc = jnp.clip(r, 0, ROWS_PER_TILE - 1)
        plsc.store_scatter(rows_v, [rc * NUM_PROTOTYPES + i], ones, mask=m)
        plsc.store_scatter(base_v, [rc], zeros, mask=m)
        return carry

    lax.fori_loop(start // L, (end + L - 1) // L, scan_body, 0)

    mn = jnp.minimum(jnp.min(base_v[pl.ds(0, L)]), jnp.min(base_v[pl.ds(L, L)]))

    @pl.when(mn < 0.0)
    def _fix():
        def fix_body(c, carry):
            row = c // COLS_CHUNKS
            bvec = plsc.load_gather(base_v, [jnp.full((L,), row, jnp.int32)])
            chunk = rows_v[pl.ds(c * L, L)]
            rows_v[pl.ds(c * L, L)] = chunk + bvec
            return carry

        lax.fori_loop(0, OUT_PER_TILE // L, fix_body, 0)

    pltpu.sync_copy(rows_v, out_hbm.at[pl.ds(wid * OUT_PER_TILE, OUT_PER_TILE)])


def kernel(le_embeddings, belonging, prototype_vectors):
    idx = _assign(le_embeddings, prototype_vectors).reshape(N_TOKENS)
    bel = belonging.astype(jnp.int32)
    zero = jnp.zeros((OUT_PER_TILE,), jnp.float32)
    out = _make_scatter()(zero, bel, idx)
    return out.reshape(N_GRAPHS, NUM_PROTOTYPES)


# speedup vs baseline: 1.0360x; 1.0360x over previous
import functools

import jax
import jax.numpy as jnp
from jax import lax
from jax.experimental import pallas as pl
from jax.experimental.pallas import tpu as pltpu
from jax.experimental.pallas import tpu_sc as plsc

NUM_PROTOTYPES = 1024
DIM = 32
N_TOKENS = 16384
N_GRAPHS = 1024

TOK_BLOCK = 2048
N_BLOCKS = N_TOKENS // TOK_BLOCK


def _assign_body(e_ref, c_ref, idx_ref):
    e = e_ref[...]
    c = c_ref[...]
    a2 = jnp.sum(e * e, axis=1, keepdims=True)
    b2 = jnp.sum(c * c, axis=1)[None, :]
    prod2 = lax.dot_general(e, c * -2.0, (((1,), (1,)), ((), ())),
                            preferred_element_type=jnp.float32)
    d2 = (a2 + b2) + prod2
    m2 = jnp.maximum(jnp.min(d2, axis=1, keepdims=True), 1e-12)
    dmin = jnp.sqrt(m2)
    mi = lax.bitcast_convert_type(m2, jnp.int32)
    B = m2
    for k in range(1, 6):
        cand = lax.bitcast_convert_type(mi + k, jnp.float32)
        B = jnp.where(jnp.sqrt(cand) == dmin, cand, B)
    z = jnp.maximum(d2, B)
    idx = jnp.argmin(z, axis=1).astype(jnp.int32)
    idx_ref[0, 0, :] = idx


def _assign(le_embeddings, prototype_vectors):
    return pl.pallas_call(
        _assign_body,
        grid=(N_BLOCKS,),
        in_specs=[
            pl.BlockSpec((TOK_BLOCK, DIM), lambda i: (i, 0)),
            pl.BlockSpec((NUM_PROTOTYPES, DIM), lambda i: (0, 0)),
        ],
        out_specs=pl.BlockSpec((1, 1, TOK_BLOCK), lambda i: (i, 0, 0)),
        out_shape=jax.ShapeDtypeStruct((N_BLOCKS, 1, TOK_BLOCK), jnp.int32),
    )(le_embeddings, prototype_vectors)


L = 16
N_WORKERS = 32
ROWS_PER_TILE = N_GRAPHS // N_WORKERS
OUT_PER_TILE = ROWS_PER_TILE * NUM_PROTOTYPES
N_CHUNKS = N_TOKENS // L
COLS_CHUNKS = NUM_PROTOTYPES // L

@functools.cache
def _make_scatter():
    mesh = plsc.VectorSubcoreMesh(core_axis_name="c", subcore_axis_name="s")
    return functools.partial(
        pl.kernel,
        mesh=mesh,
        compiler_params=pltpu.CompilerParams(needs_layout_passes=False),
        out_type=jax.ShapeDtypeStruct((N_GRAPHS * NUM_PROTOTYPES,), jnp.float32),
        scratch_types=[
            pltpu.VMEM((N_TOKENS,), jnp.int32),
            pltpu.VMEM((N_TOKENS,), jnp.int32),
            pltpu.VMEM((OUT_PER_TILE,), jnp.float32),
            pltpu.VMEM((ROWS_PER_TILE,), jnp.float32),
        ],
    )(_scatter_body)


def _lower_bounds(bel_v, targets):
    def body(k, lo):
        nxt = lo + (jnp.int32(N_TOKENS) >> k)
        idx = jnp.minimum(nxt, N_TOKENS - 1)
        v = plsc.load_gather(bel_v, [idx])
        take = (nxt <= N_TOKENS - 1) & (v < targets)
        return jnp.where(take, nxt, lo)

    lo0 = jnp.full((L,), -1, jnp.int32)
    return lax.fori_loop(0, 15, body, lo0) + 1


def _scatter_body(zero_hbm, bel_hbm, idx_hbm, out_hbm, bel_v, idx_v, rows_v,
                  base_v):
    wid = lax.axis_index("s") * 2 + lax.axis_index("c")
    g0 = wid * ROWS_PER_TILE

    pltpu.sync_copy(zero_hbm, rows_v)
    pltpu.sync_copy(bel_hbm, bel_v)
    pltpu.sync_copy(idx_hbm, idx_v)

    zeros = jnp.zeros((L,), jnp.float32)
    ones = jnp.ones((L,), jnp.float32)
    neg = jnp.full((L,), -jnp.inf, jnp.float32)

    base_v[pl.ds(0, L)] = neg
    base_v[pl.ds(L, L)] = neg

    lanes = lax.iota(jnp.int32, 16)
    bounds = _lower_bounds(bel_v, g0 + lanes * ROWS_PER_TILE)
    start = jnp.max(jnp.where(lanes == 0, bounds, 0))
    end = jnp.max(jnp.where(lanes == 1, bounds, 0))

    def scan_body(c, carry):
        b = bel_v[pl.ds(c * L, L)]
        i = idx_v[pl.ds(c * L, L)]
        r = b - g0
        m = (r >= 0) & (r < ROWS_PER_TILE)
        rc = jnp.clip(r, 0, ROWS_PER_TILE - 1)
        plsc.store_scatter(rows_v, [rc * NUM_PROTOTYPES + i], ones, mask=m)
        plsc.store_scatter(base_v, [rc], zeros, mask=m)
        return carry

    lax.fori_loop(start // L, (end + L - 1) // L, scan_body, 0)

    mn = jnp.minimum(jnp.min(base_v[pl.ds(0, L)]), jnp.min(base_v[pl.ds(L, L)]))

    @pl.when(mn < 0.0)
    def _fix():
        def fix_body(c, carry):
            row = c // COLS_CHUNKS
            bvec = plsc.load_gather(base_v, [jnp.full((L,), row, jnp.int32)])
            chunk = rows_v[pl.ds(c * L, L)]
            rows_v[pl.ds(c * L, L)] = chunk + bvec
            return carry

        lax.fori_loop(0, OUT_PER_TILE // L, fix_body, 0)

    pltpu.sync_copy(rows_v, out_hbm.at[pl.ds(wid * OUT_PER_TILE, OUT_PER_TILE)])


def kernel(le_embeddings, belonging, prototype_vectors):
    idx = _assign(le_embeddings, prototype_vectors).reshape(N_TOKENS)
    bel = belonging.astype(jnp.int32)
    zero = jnp.zeros((OUT_PER_TILE,), jnp.float32)
    out = _make_scatter()(zero, bel, idx)
    return out.reshape(N_GRAPHS, NUM_PROTOTYPES)
